# baseline (speedup 1.0000x reference)
import functools
import math

import jax
import jax.numpy as jnp
from jax import lax
from jax.experimental import pallas as pl
from jax.experimental.pallas import tpu as pltpu
from jax.experimental.pallas import tpu_sc as plsc

D_MODEL = 64
SCALE = math.sqrt(D_MODEL)

NUM_CORES = 2
NUM_SUBCORES = 16
NUM_WORKERS = NUM_CORES * NUM_SUBCORES

CHUNK = 128
NBUF = 2


def _emb_body(x_hbm, table_hbm, out_hbm, idx_v, in0, in1, ou0, ou1,
              gs0, gs1, os0, os1, *, per_w, n_chunks):
    wid = lax.axis_index("s") * NUM_CORES + lax.axis_index("c")
    base = wid * per_w
    ins, outs = (in0, in1), (ou0, ou1)
    gsems, osems = (gs0, gs1), (os0, os1)

    pltpu.sync_copy(x_hbm.at[pl.ds(base, per_w)], idx_v)

    def idx_slice(ci):
        off = pl.multiple_of(ci * CHUNK, CHUNK)
        return idx_v.at[pl.ds(off, CHUNK)]

    def out_slice(off):
        return out_hbm.at[pl.ds(off, CHUNK), pl.ds(0, D_MODEL)]

    for b in range(NBUF):
        pltpu.async_copy(table_hbm.at[idx_slice(b)], ins[b], gsems[b])

    def group(g, carry):
        for b in range(NBUF):
            ci = g * NBUF + b
            pltpu.make_async_copy(table_hbm.at[idx_slice(ci)], ins[b],
                                  gsems[b]).wait()

            @pl.when(g > 0)
            def _():
                pltpu.make_async_copy(outs[b], out_slice(base), osems[b]).wait()

            @plsc.parallel_loop(0, CHUNK, unroll=8)
            def _(j):
                for k in range(D_MODEL // 16):
                    sl = pl.ds(k * 16, 16)
                    outs[b][j, sl] = ins[b][j, sl] * SCALE

            pltpu.async_copy(outs[b], out_slice(base + ci * CHUNK), osems[b])

            @pl.when(ci < n_chunks - NBUF)
            def _():
                pltpu.async_copy(table_hbm.at[idx_slice(ci + NBUF)], ins[b],
                                 gsems[b])
        return carry

    lax.fori_loop(0, n_chunks // NBUF, group, 0)

    for b in range(NBUF):
        pltpu.make_async_copy(outs[b], out_slice(base), osems[b]).wait()


def kernel(x, table):
    b0, b1 = x.shape
    n_total = b0 * b1
    assert n_total % (NUM_WORKERS * CHUNK * NBUF) == 0
    per_w = n_total // NUM_WORKERS
    n_chunks = per_w // CHUNK

    mesh = plsc.VectorSubcoreMesh(core_axis_name="c", subcore_axis_name="s")
    emb = functools.partial(
        pl.kernel,
        mesh=mesh,
        out_type=jax.ShapeDtypeStruct((n_total, 2 * D_MODEL), jnp.float32),
        scratch_types=[
            pltpu.VMEM((per_w,), jnp.int32),
            pltpu.VMEM((CHUNK, D_MODEL), jnp.float32),
            pltpu.VMEM((CHUNK, D_MODEL), jnp.float32),
            pltpu.VMEM((CHUNK, D_MODEL), jnp.float32),
            pltpu.VMEM((CHUNK, D_MODEL), jnp.float32),
            pltpu.SemaphoreType.DMA,
            pltpu.SemaphoreType.DMA,
            pltpu.SemaphoreType.DMA,
            pltpu.SemaphoreType.DMA,
        ],
        compiler_params=pltpu.CompilerParams(use_tc_tiling_on_sc=False),
    )(functools.partial(_emb_body, per_w=per_w, n_chunks=n_chunks))

    x_flat = jnp.bitwise_and(x, 0x3FFFFF).reshape(n_total) * 2
    vocab = table.shape[0]
    table2 = jnp.pad(table[:, None, :], ((0, 0), (0, 1), (0, 0))).reshape(
        2 * vocab, D_MODEL)
    out = emb(x_flat, table2)
    return out[:, :D_MODEL].reshape(b0, b1, D_MODEL)


# speedup vs baseline: 1.8978x; 1.8978x over previous
import functools
import math

import jax
import jax.numpy as jnp
from jax import lax
from jax.experimental import pallas as pl
from jax.experimental.pallas import tpu as pltpu
from jax.experimental.pallas import tpu_sc as plsc

D_MODEL = 64
SCALE = math.sqrt(D_MODEL)

NUM_CORES = 2
NUM_SUBCORES = 16
NUM_WORKERS = NUM_CORES * NUM_SUBCORES

CHUNK = 128
NBUF = 2


def _emb_body(x_hbm, table_hbm, out_hbm, idx_v, in0, in1, ou0, ou1,
              gs0, gs1, os0, os1, *, per_w, n_chunks):
    wid = lax.axis_index("s") * NUM_CORES + lax.axis_index("c")
    base = wid * per_w
    ins, outs = (in0, in1), (ou0, ou1)
    gsems, osems = (gs0, gs1), (os0, os1)

    pltpu.sync_copy(x_hbm.at[pl.ds(base, per_w)], idx_v)

    def idx_slice(ci):
        off = pl.multiple_of(ci * CHUNK, CHUNK)
        return idx_v.at[pl.ds(off, CHUNK)]

    def out_slice(off):
        return out_hbm.at[pl.ds(off, CHUNK), pl.ds(0, D_MODEL)]

    for b in range(NBUF):
        pltpu.async_copy(table_hbm.at[idx_slice(b)], ins[b], gsems[b])

    def group(g, carry):
        for b in range(NBUF):
            ci = g * NBUF + b
            pltpu.make_async_copy(table_hbm.at[idx_slice(ci)], ins[b],
                                  gsems[b]).wait()

            @pl.when(g > 0)
            def _():
                pltpu.make_async_copy(outs[b], out_slice(base), osems[b]).wait()

            @plsc.parallel_loop(0, CHUNK, unroll=8)
            def _(j):
                for k in range(D_MODEL // 16):
                    sl = pl.ds(k * 16, 16)
                    outs[b][j, sl] = ins[b][j, sl] * SCALE

            pltpu.async_copy(outs[b], out_slice(base + ci * CHUNK), osems[b])

            @pl.when(ci < n_chunks - NBUF)
            def _():
                pltpu.async_copy(table_hbm.at[idx_slice(ci + NBUF)], ins[b],
                                 gsems[b])
        return carry

    lax.fori_loop(0, n_chunks // NBUF, group, 0)

    for b in range(NBUF):
        pltpu.make_async_copy(outs[b], out_slice(base), osems[b]).wait()


REPACK_BV = 2048


def _repack_body(t_ref, o_ref):
    blk = t_ref[...]
    half = REPACK_BV // 2
    r = lax.broadcasted_iota(jnp.int32, (D_MODEL, 2 * D_MODEL), 0)
    c = lax.broadcasted_iota(jnp.int32, (D_MODEL, 2 * D_MODEL), 1)
    p_lo = (c == r).astype(jnp.float32)
    p_hi = (c == r + D_MODEL).astype(jnp.float32)
    dn = (((0,), (0,)), ((), ()))
    o_ref[...] = (
        lax.dot_general(blk[:, :half], p_lo, dn,
                        precision=lax.Precision.HIGHEST)
        + lax.dot_general(blk[:, half:], p_hi, dn,
                          precision=lax.Precision.HIGHEST))


def _repack_table(table):
    vocab, d = table.shape
    grid = -(-vocab // REPACK_BV)
    rep = pl.pallas_call(
        _repack_body,
        grid=(grid,),
        in_specs=[pl.BlockSpec((d, REPACK_BV), lambda g: (0, g))],
        out_specs=pl.BlockSpec((REPACK_BV // 2, 2 * d), lambda g: (g, 0)),
        out_shape=jax.ShapeDtypeStruct((grid * REPACK_BV // 2, 2 * d),
                                       jnp.float32),
    )
    return rep(table.T).reshape(grid * REPACK_BV, d)


def kernel(x, table):
    b0, b1 = x.shape
    n_total = b0 * b1
    assert n_total % (NUM_WORKERS * CHUNK * NBUF) == 0
    per_w = n_total // NUM_WORKERS
    n_chunks = per_w // CHUNK

    mesh = plsc.VectorSubcoreMesh(core_axis_name="c", subcore_axis_name="s")
    emb = functools.partial(
        pl.kernel,
        mesh=mesh,
        out_type=jax.ShapeDtypeStruct((n_total, 2 * D_MODEL), jnp.float32),
        scratch_types=[
            pltpu.VMEM((per_w,), jnp.int32),
            pltpu.VMEM((CHUNK, D_MODEL), jnp.float32),
            pltpu.VMEM((CHUNK, D_MODEL), jnp.float32),
            pltpu.VMEM((CHUNK, D_MODEL), jnp.float32),
            pltpu.VMEM((CHUNK, D_MODEL), jnp.float32),
            pltpu.SemaphoreType.DMA,
            pltpu.SemaphoreType.DMA,
            pltpu.SemaphoreType.DMA,
            pltpu.SemaphoreType.DMA,
        ],
        compiler_params=pltpu.CompilerParams(use_tc_tiling_on_sc=False),
    )(functools.partial(_emb_body, per_w=per_w, n_chunks=n_chunks))

    v = jnp.bitwise_and(x, 0x3FFFFF).reshape(n_total)
    x_flat = ((v & ~(REPACK_BV - 1)) + ((v & (REPACK_BV // 2 - 1)) << 1)
              + ((v >> 10) & 1))
    out = emb(x_flat, _repack_table(table))
    return out[:, :D_MODEL].reshape(b0, b1, D_MODEL)


# speedup vs baseline: 2.4789x; 1.3062x over previous
import functools
import math

import jax
import jax.numpy as jnp
from jax import lax
from jax.experimental import pallas as pl
from jax.experimental.pallas import tpu as pltpu
from jax.experimental.pallas import tpu_sc as plsc

D_MODEL = 64
SCALE = math.sqrt(D_MODEL)

NUM_CORES = 2
NUM_SUBCORES = 16
NUM_WORKERS = NUM_CORES * NUM_SUBCORES

CHUNK = 128
NBUF = 2


def _emb_body(x_hbm, table_hbm, out_hbm, idx_v, in0, in1, ou0, ou1,
              gs0, gs1, os0, os1, *, per_w, n_chunks):
    wid = lax.axis_index("s") * NUM_CORES + lax.axis_index("c")
    base = wid * per_w
    ins, outs = (in0, in1), (ou0, ou1)
    gsems, osems = (gs0, gs1), (os0, os1)

    pltpu.sync_copy(x_hbm.at[pl.ds(base, per_w)], idx_v)

    def idx_slice(ci):
        off = pl.multiple_of(ci * CHUNK, CHUNK)
        return idx_v.at[pl.ds(off, CHUNK)]

    def out_slice(off):
        return out_hbm.at[pl.ds(off, CHUNK), pl.ds(0, D_MODEL)]

    for b in range(NBUF):
        pltpu.async_copy(table_hbm.at[idx_slice(b)], ins[b], gsems[b])

    def group(g, carry):
        for b in range(NBUF):
            ci = g * NBUF + b
            pltpu.make_async_copy(table_hbm.at[idx_slice(ci)], ins[b],
                                  gsems[b]).wait()

            @pl.when(g > 0)
            def _():
                pltpu.make_async_copy(outs[b], out_slice(base), osems[b]).wait()

            @plsc.parallel_loop(0, CHUNK, unroll=8)
            def _(j):
                for k in range(D_MODEL // 16):
                    sl = pl.ds(k * 16, 16)
                    outs[b][j, sl] = ins[b][j, sl] * SCALE

            pltpu.async_copy(outs[b], out_slice(base + ci * CHUNK), osems[b])

            @pl.when(ci < n_chunks - NBUF)
            def _():
                pltpu.async_copy(table_hbm.at[idx_slice(ci + NBUF)], ins[b],
                                 gsems[b])
        return carry

    lax.fori_loop(0, n_chunks // NBUF, group, 0)

    for b in range(NBUF):
        pltpu.make_async_copy(outs[b], out_slice(base), osems[b]).wait()


REPACK_BV = 2048


def _repack_body(t_ref, o_ref):
    blk = t_ref[...]
    half = REPACK_BV // 2
    stacked = jnp.concatenate([blk[:, :half], blk[:, half:]], axis=0)
    o_ref[...] = jnp.transpose(stacked)


def _repack_table(table):
    vocab, d = table.shape
    grid = -(-vocab // REPACK_BV)
    rep = pl.pallas_call(
        _repack_body,
        grid=(grid,),
        in_specs=[pl.BlockSpec((d, REPACK_BV), lambda g: (0, g))],
        out_specs=pl.BlockSpec((REPACK_BV // 2, 2 * d), lambda g: (g, 0)),
        out_shape=jax.ShapeDtypeStruct((grid * REPACK_BV // 2, 2 * d),
                                       jnp.float32),
    )
    return rep(table.T).reshape(grid * REPACK_BV, d)


def kernel(x, table):
    b0, b1 = x.shape
    n_total = b0 * b1
    assert n_total % (NUM_WORKERS * CHUNK * NBUF) == 0
    per_w = n_total // NUM_WORKERS
    n_chunks = per_w // CHUNK

    mesh = plsc.VectorSubcoreMesh(core_axis_name="c", subcore_axis_name="s")
    emb = functools.partial(
        pl.kernel,
        mesh=mesh,
        out_type=jax.ShapeDtypeStruct((n_total, 2 * D_MODEL), jnp.float32),
        scratch_types=[
            pltpu.VMEM((per_w,), jnp.int32),
            pltpu.VMEM((CHUNK, D_MODEL), jnp.float32),
            pltpu.VMEM((CHUNK, D_MODEL), jnp.float32),
            pltpu.VMEM((CHUNK, D_MODEL), jnp.float32),
            pltpu.VMEM((CHUNK, D_MODEL), jnp.float32),
            pltpu.SemaphoreType.DMA,
            pltpu.SemaphoreType.DMA,
            pltpu.SemaphoreType.DMA,
            pltpu.SemaphoreType.DMA,
        ],
        compiler_params=pltpu.CompilerParams(use_tc_tiling_on_sc=False),
    )(functools.partial(_emb_body, per_w=per_w, n_chunks=n_chunks))

    v = jnp.bitwise_and(x, 0x3FFFFF).reshape(n_total)
    x_flat = ((v & ~(REPACK_BV - 1)) + ((v & (REPACK_BV // 2 - 1)) << 1)
              + ((v >> 10) & 1))
    out = emb(x_flat, _repack_table(table))
    return out[:, :D_MODEL].reshape(b0, b1, D_MODEL)


# speedup vs baseline: 2.9089x; 1.1734x over previous
import functools
import math

import jax
import jax.numpy as jnp
from jax import lax
from jax.experimental import pallas as pl
from jax.experimental.pallas import tpu as pltpu
from jax.experimental.pallas import tpu_sc as plsc

D_MODEL = 64
SCALE = math.sqrt(D_MODEL)

NUM_CORES = 2
NUM_SUBCORES = 16
NUM_WORKERS = NUM_CORES * NUM_SUBCORES

CHUNK = 128
NBUF = 2


def _emb_body(x_hbm, table_hbm, out_hbm, idx_v, in0, in1, ou0, ou1,
              gs0, gs1, os0, os1, *, per_w, n_chunks):
    wid = lax.axis_index("s") * NUM_CORES + lax.axis_index("c")
    base = wid * per_w
    ins, outs = (in0, in1), (ou0, ou1)
    gsems, osems = (gs0, gs1), (os0, os1)

    pltpu.sync_copy(x_hbm.at[pl.ds(base, per_w)], idx_v)

    def idx_slice(ci):
        off = pl.multiple_of(ci * CHUNK, CHUNK)
        return idx_v.at[pl.ds(off, CHUNK)]

    def out_slice(off):
        return out_hbm.at[pl.ds(off, CHUNK), pl.ds(0, D_MODEL)]

    for b in range(NBUF):
        pltpu.async_copy(table_hbm.at[idx_slice(b)], ins[b], gsems[b])

    def group(g, carry):
        for b in range(NBUF):
            ci = g * NBUF + b
            pltpu.make_async_copy(table_hbm.at[idx_slice(ci)], ins[b],
                                  gsems[b]).wait()

            @pl.when(g > 0)
            def _():
                pltpu.make_async_copy(outs[b], out_slice(base), osems[b]).wait()

            @plsc.parallel_loop(0, CHUNK, unroll=8)
            def _(j):
                for k in range(D_MODEL // 16):
                    sl = pl.ds(k * 16, 16)
                    outs[b][j, sl] = ins[b][j, sl] * SCALE

            pltpu.async_copy(outs[b], out_slice(base + ci * CHUNK), osems[b])

            @pl.when(ci < n_chunks - NBUF)
            def _():
                pltpu.async_copy(table_hbm.at[idx_slice(ci + NBUF)], ins[b],
                                 gsems[b])
        return carry

    lax.fori_loop(0, n_chunks // NBUF, group, 0)

    for b in range(NBUF):
        pltpu.make_async_copy(outs[b], out_slice(base), osems[b]).wait()


REPACK_BV = 4096


def _repack_body(t_ref, o_ref):
    blk = t_ref[...]
    half = REPACK_BV // 2
    stacked = jnp.concatenate([blk[:, :half], blk[:, half:]], axis=0)
    o_ref[...] = jnp.transpose(stacked)


def _repack_table(table):
    vocab, d = table.shape
    grid = -(-vocab // REPACK_BV)
    rep = pl.pallas_call(
        _repack_body,
        grid=(grid,),
        in_specs=[pl.BlockSpec((d, REPACK_BV), lambda g: (0, g))],
        out_specs=pl.BlockSpec((REPACK_BV // 2, 2 * d), lambda g: (g, 0)),
        out_shape=jax.ShapeDtypeStruct((grid * REPACK_BV // 2, 2 * d),
                                       jnp.float32),
    )
    return rep(table.T).reshape(grid * REPACK_BV, d)


def kernel(x, table):
    b0, b1 = x.shape
    n_total = b0 * b1
    assert n_total % (NUM_WORKERS * CHUNK * NBUF) == 0
    per_w = n_total // NUM_WORKERS
    n_chunks = per_w // CHUNK

    mesh = plsc.VectorSubcoreMesh(core_axis_name="c", subcore_axis_name="s")
    emb = functools.partial(
        pl.kernel,
        mesh=mesh,
        out_type=jax.ShapeDtypeStruct((n_total, 2 * D_MODEL), jnp.float32),
        scratch_types=[
            pltpu.VMEM((per_w,), jnp.int32),
            pltpu.VMEM((CHUNK, D_MODEL), jnp.float32),
            pltpu.VMEM((CHUNK, D_MODEL), jnp.float32),
            pltpu.VMEM((CHUNK, D_MODEL), jnp.float32),
            pltpu.VMEM((CHUNK, D_MODEL), jnp.float32),
            pltpu.SemaphoreType.DMA,
            pltpu.SemaphoreType.DMA,
            pltpu.SemaphoreType.DMA,
            pltpu.SemaphoreType.DMA,
        ],
        compiler_params=pltpu.CompilerParams(use_tc_tiling_on_sc=False),
    )(functools.partial(_emb_body, per_w=per_w, n_chunks=n_chunks))

    v = jnp.bitwise_and(x, 0x3FFFFF).reshape(n_total)
    half_shift = (REPACK_BV // 2).bit_length() - 1
    x_flat = ((v & ~(REPACK_BV - 1)) + ((v & (REPACK_BV // 2 - 1)) << 1)
              + ((v >> half_shift) & 1))
    out = emb(x_flat, _repack_table(table))
    return out[:, :D_MODEL].reshape(b0, b1, D_MODEL)


# speedup vs baseline: 3.3037x; 1.1357x over previous
import functools
import math

import jax
import jax.numpy as jnp
from jax import lax
from jax.experimental import pallas as pl
from jax.experimental.pallas import tpu as pltpu
from jax.experimental.pallas import tpu_sc as plsc

D_MODEL = 64
SCALE = math.sqrt(D_MODEL)

NUM_CORES = 2
NUM_SUBCORES = 16
NUM_WORKERS = NUM_CORES * NUM_SUBCORES

CHUNK = 128
NBUF = 2


def _emb_body(x_hbm, table_hbm, out_hbm, idx_v, in0, in1, ou0, ou1,
              gs0, gs1, os0, os1, *, per_w, n_chunks):
    wid = lax.axis_index("s") * NUM_CORES + lax.axis_index("c")
    base = wid * per_w
    ins, outs = (in0, in1), (ou0, ou1)
    gsems, osems = (gs0, gs1), (os0, os1)

    pltpu.sync_copy(x_hbm.at[pl.ds(base, per_w)], idx_v)

    def idx_slice(ci):
        off = pl.multiple_of(ci * CHUNK, CHUNK)
        return idx_v.at[pl.ds(off, CHUNK)]

    def out_slice(off):
        return out_hbm.at[pl.ds(off, CHUNK), pl.ds(0, D_MODEL)]

    for b in range(NBUF):
        pltpu.async_copy(table_hbm.at[idx_slice(b)], ins[b], gsems[b])

    def group(g, carry):
        for b in range(NBUF):
            ci = g * NBUF + b
            pltpu.make_async_copy(table_hbm.at[idx_slice(ci)], ins[b],
                                  gsems[b]).wait()

            @pl.when(g > 0)
            def _():
                pltpu.make_async_copy(outs[b], out_slice(base), osems[b]).wait()

            @plsc.parallel_loop(0, CHUNK, unroll=8)
            def _(j):
                for k in range(D_MODEL // 16):
                    sl = pl.ds(k * 16, 16)
                    outs[b][j, sl] = ins[b][j, sl] * SCALE

            pltpu.async_copy(outs[b], out_slice(base + ci * CHUNK), osems[b])

            @pl.when(ci < n_chunks - NBUF)
            def _():
                pltpu.async_copy(table_hbm.at[idx_slice(ci + NBUF)], ins[b],
                                 gsems[b])
        return carry

    lax.fori_loop(0, n_chunks // NBUF, group, 0)

    for b in range(NBUF):
        pltpu.make_async_copy(outs[b], out_slice(base), osems[b]).wait()


REPACK_BV = 8192


def _repack_body(t_ref, o_ref):
    blk = t_ref[...]
    half = REPACK_BV // 2
    stacked = jnp.concatenate([blk[:, :half], blk[:, half:]], axis=0)
    o_ref[...] = jnp.transpose(stacked)


def _repack_table(table):
    vocab, d = table.shape
    grid = -(-vocab // REPACK_BV)
    rep = pl.pallas_call(
        _repack_body,
        grid=(grid,),
        in_specs=[pl.BlockSpec((d, REPACK_BV), lambda g: (0, g))],
        out_specs=pl.BlockSpec((REPACK_BV // 2, 2 * d), lambda g: (g, 0)),
        out_shape=jax.ShapeDtypeStruct((grid * REPACK_BV // 2, 2 * d),
                                       jnp.float32),
    )
    return rep(table.T).reshape(grid * REPACK_BV, d)


def kernel(x, table):
    b0, b1 = x.shape
    n_total = b0 * b1
    assert n_total % (NUM_WORKERS * CHUNK * NBUF) == 0
    per_w = n_total // NUM_WORKERS
    n_chunks = per_w // CHUNK

    mesh = plsc.VectorSubcoreMesh(core_axis_name="c", subcore_axis_name="s")
    emb = functools.partial(
        pl.kernel,
        mesh=mesh,
        out_type=jax.ShapeDtypeStruct((n_total, 2 * D_MODEL), jnp.float32),
        scratch_types=[
            pltpu.VMEM((per_w,), jnp.int32),
            pltpu.VMEM((CHUNK, D_MODEL), jnp.float32),
            pltpu.VMEM((CHUNK, D_MODEL), jnp.float32),
            pltpu.VMEM((CHUNK, D_MODEL), jnp.float32),
            pltpu.VMEM((CHUNK, D_MODEL), jnp.float32),
            pltpu.SemaphoreType.DMA,
            pltpu.SemaphoreType.DMA,
            pltpu.SemaphoreType.DMA,
            pltpu.SemaphoreType.DMA,
        ],
        compiler_params=pltpu.CompilerParams(use_tc_tiling_on_sc=False),
    )(functools.partial(_emb_body, per_w=per_w, n_chunks=n_chunks))

    v = jnp.bitwise_and(x, 0x3FFFFF).reshape(n_total)
    half_shift = (REPACK_BV // 2).bit_length() - 1
    x_flat = ((v & ~(REPACK_BV - 1)) + ((v & (REPACK_BV // 2 - 1)) << 1)
              + ((v >> half_shift) & 1))
    out = emb(x_flat, _repack_table(table))
    return out[:, :D_MODEL].reshape(b0, b1, D_MODEL)
